# baseline (speedup 1.0000x reference)
import functools

import jax
import jax.numpy as jnp
from jax import lax
from jax.experimental import pallas as pl
from jax.experimental.pallas import tpu as pltpu
from jax.experimental.pallas import tpu_sc as plsc


def _sc_gather_rows(labels, gammas, betas):
    B = labels.shape[0]
    C = gammas.shape[1]
    info = plsc.get_sparse_core_info()

    b_per_w = 8
    n_active = B // b_per_w
    assert B % b_per_w == 0

    mesh = plsc.VectorSubcoreMesh(core_axis_name="c", subcore_axis_name="s")

    @functools.partial(
        pl.kernel,
        mesh=mesh,
        out_type=jax.ShapeDtypeStruct((2 * B, C), jnp.float32),
        scratch_types=[
            pltpu.VMEM((b_per_w,), jnp.int32),
            pltpu.VMEM((b_per_w, C), jnp.float32),
            pltpu.VMEM((b_per_w, C), jnp.float32),
            pltpu.SemaphoreType.DMA,
            pltpu.SemaphoreType.DMA,
        ],
    )
    def gather_kernel(labels_hbm, gammas_hbm, betas_hbm, out_hbm,
                      idx_v, grows_v, brows_v, gsem, bsem):
        wid = lax.axis_index("s") * info.num_cores + lax.axis_index("c")

        @pl.when(wid < n_active)
        def _():
            base = wid * b_per_w
            pltpu.sync_copy(labels_hbm.at[pl.ds(base, b_per_w)], idx_v)
            g_cp = pltpu.async_copy(gammas_hbm.at[idx_v], grows_v, gsem)
            b_cp = pltpu.async_copy(betas_hbm.at[idx_v], brows_v, bsem)
            g_cp.wait()
            b_cp.wait()
            g_out = pltpu.async_copy(grows_v, out_hbm.at[pl.ds(base, b_per_w)], gsem)
            b_out = pltpu.async_copy(brows_v, out_hbm.at[pl.ds(B + base, b_per_w)], bsem)
            g_out.wait()
            b_out.wait()

    return gather_kernel(labels, gammas, betas)


def _make_film_body(B, nb):
    def _film_body(x_ref, gb_hbm, o_ref, gb_vmem, sem):
        i = pl.program_id(0)

        @pl.when(i == 0)
        def _():
            cp = pltpu.make_async_copy(gb_hbm, gb_vmem, sem)
            cp.start()
            cp.wait()

        for j in range(nb):
            g = gb_vmem[i * nb + j, :]
            b = gb_vmem[B + i * nb + j, :]
            o_ref[j, :, :] = x_ref[j, :, :] * g[None, :] + b[None, :]

    return _film_body


def _film(batch3, gbrows, nb=4):
    B, P, C = batch3.shape
    return pl.pallas_call(
        _make_film_body(B, nb),
        grid=(B // nb,),
        in_specs=[
            pl.BlockSpec((nb, P, C), lambda i: (i, 0, 0)),
            pl.BlockSpec(memory_space=pl.ANY),
        ],
        out_specs=pl.BlockSpec((nb, P, C), lambda i: (i, 0, 0)),
        out_shape=jax.ShapeDtypeStruct((B, P, C), jnp.float32),
        scratch_shapes=[
            pltpu.VMEM((2 * B, C), jnp.float32),
            pltpu.SemaphoreType.DMA,
        ],
        compiler_params=pltpu.CompilerParams(
            dimension_semantics=("arbitrary",),
        ),
    )(batch3, gbrows)


def kernel(batch, labels, gammas, betas):
    B, C, H, W = batch.shape
    labels = labels.astype(jnp.int32)
    gbrows = _sc_gather_rows(labels, gammas, betas)
    fake = jnp.concatenate([gammas[:B], betas[:B]], axis=0)
    bt = jnp.transpose(batch, (0, 2, 3, 1)).reshape(B, H * W, C)
    out = _film(bt, fake)
    return jnp.transpose(out.reshape(B, H, W, C), (0, 3, 1, 2)), gbrows


# speedup vs baseline: 1.0127x; 1.0127x over previous
import functools

import jax
import jax.numpy as jnp
from jax import lax
from jax.experimental import pallas as pl
from jax.experimental.pallas import tpu as pltpu
from jax.experimental.pallas import tpu_sc as plsc


def _sc_gather_rows(labels, gammas, betas):
    B = labels.shape[0]
    C = gammas.shape[1]
    info = plsc.get_sparse_core_info()

    b_per_w = 8
    n_active = B // b_per_w
    assert B % b_per_w == 0

    mesh = plsc.VectorSubcoreMesh(core_axis_name="c", subcore_axis_name="s")

    @functools.partial(
        pl.kernel,
        mesh=mesh,
        out_type=[
            jax.ShapeDtypeStruct((B, C), jnp.float32),
            jax.ShapeDtypeStruct((B, C), jnp.float32),
        ],
        scratch_types=[
            pltpu.VMEM((b_per_w,), jnp.int32),
            pltpu.VMEM((b_per_w, C), jnp.float32),
            pltpu.VMEM((b_per_w, C), jnp.float32),
            pltpu.SemaphoreType.DMA,
            pltpu.SemaphoreType.DMA,
        ],
    )
    def gather_kernel(labels_hbm, gammas_hbm, betas_hbm, gout_hbm, bout_hbm,
                      idx_v, grows_v, brows_v, gsem, bsem):
        wid = lax.axis_index("s") * info.num_cores + lax.axis_index("c")

        @pl.when(wid < n_active)
        def _():
            base = wid * b_per_w
            pltpu.sync_copy(labels_hbm.at[pl.ds(base, b_per_w)], idx_v)
            g_cp = pltpu.async_copy(gammas_hbm.at[idx_v], grows_v, gsem)
            b_cp = pltpu.async_copy(betas_hbm.at[idx_v], brows_v, bsem)
            g_cp.wait()
            b_cp.wait()
            g_out = pltpu.async_copy(grows_v, gout_hbm.at[pl.ds(base, b_per_w)], gsem)
            b_out = pltpu.async_copy(brows_v, bout_hbm.at[pl.ds(base, b_per_w)], bsem)
            g_out.wait()
            b_out.wait()

    return gather_kernel(labels, gammas, betas)


def _film_body(x_ref, g_ref, b_ref, o_ref):
    nb = x_ref.shape[0]
    i = pl.program_id(0)
    for j in range(nb):
        g = g_ref[i * nb + j, :]
        b = b_ref[i * nb + j, :]
        o_ref[j, :, :] = x_ref[j, :, :] * g[None, :] + b[None, :]


def _film(batch3, grows, brows, nb=8):
    B, P, C = batch3.shape
    return pl.pallas_call(
        _film_body,
        grid=(B // nb,),
        in_specs=[
            pl.BlockSpec((nb, P, C), lambda i: (i, 0, 0)),
            pl.BlockSpec((B, C), lambda i: (0, 0)),
            pl.BlockSpec((B, C), lambda i: (0, 0)),
        ],
        out_specs=pl.BlockSpec((nb, P, C), lambda i: (i, 0, 0)),
        out_shape=jax.ShapeDtypeStruct((B, P, C), jnp.float32),
        compiler_params=pltpu.CompilerParams(
            dimension_semantics=("arbitrary",),
        ),
    )(batch3, grows, brows)


def kernel(batch, labels, gammas, betas):
    B, C, H, W = batch.shape
    labels = labels.astype(jnp.int32)
    grows, brows = _sc_gather_rows(labels, gammas, betas)
    bt = jnp.transpose(batch, (0, 2, 3, 1)).reshape(B, H * W, C)
    out = _film(bt, grows, brows)
    return jnp.transpose(out.reshape(B, H, W, C), (0, 3, 1, 2))


# speedup vs baseline: 1.0400x; 1.0270x over previous
import functools

import jax
import jax.numpy as jnp
from jax import lax
from jax.experimental import pallas as pl
from jax.experimental.pallas import tpu as pltpu
from jax.experimental.pallas import tpu_sc as plsc


def _sc_gather_rows(labels, gammas, betas):
    B = labels.shape[0]
    C = gammas.shape[1]
    info = plsc.get_sparse_core_info()

    b_per_w = 8
    n_active = B // b_per_w
    assert B % b_per_w == 0

    mesh = plsc.VectorSubcoreMesh(core_axis_name="c", subcore_axis_name="s", num_cores=1)

    @functools.partial(
        pl.kernel,
        mesh=mesh,
        out_type=[
            jax.ShapeDtypeStruct((B, C), jnp.float32),
            jax.ShapeDtypeStruct((B, C), jnp.float32),
        ],
        scratch_types=[
            pltpu.VMEM((b_per_w,), jnp.int32),
            pltpu.VMEM((b_per_w, C), jnp.float32),
            pltpu.VMEM((b_per_w, C), jnp.float32),
            pltpu.SemaphoreType.DMA,
            pltpu.SemaphoreType.DMA,
        ],
    )
    def gather_kernel(labels_hbm, gammas_hbm, betas_hbm, gout_hbm, bout_hbm,
                      idx_v, grows_v, brows_v, gsem, bsem):
        wid = lax.axis_index("s") * 1 + lax.axis_index("c")

        @pl.when(wid < n_active)
        def _():
            base = wid * b_per_w
            pltpu.sync_copy(labels_hbm.at[pl.ds(base, b_per_w)], idx_v)
            g_cp = pltpu.async_copy(gammas_hbm.at[idx_v], grows_v, gsem)
            b_cp = pltpu.async_copy(betas_hbm.at[idx_v], brows_v, bsem)
            g_cp.wait()
            b_cp.wait()
            g_out = pltpu.async_copy(grows_v, gout_hbm.at[pl.ds(base, b_per_w)], gsem)
            b_out = pltpu.async_copy(brows_v, bout_hbm.at[pl.ds(base, b_per_w)], bsem)
            g_out.wait()
            b_out.wait()

    return gather_kernel(labels, gammas, betas)


def _film_body(x_ref, g_ref, b_ref, o_ref):
    nb = x_ref.shape[0]
    i = pl.program_id(0)
    for j in range(nb):
        g = g_ref[i * nb + j, :]
        b = b_ref[i * nb + j, :]
        o_ref[j, :, :] = x_ref[j, :, :] * g[None, :] + b[None, :]


def _film(batch3, grows, brows, nb=8):
    B, P, C = batch3.shape
    return pl.pallas_call(
        _film_body,
        grid=(B // nb,),
        in_specs=[
            pl.BlockSpec((nb, P, C), lambda i: (i, 0, 0)),
            pl.BlockSpec((B, C), lambda i: (0, 0)),
            pl.BlockSpec((B, C), lambda i: (0, 0)),
        ],
        out_specs=pl.BlockSpec((nb, P, C), lambda i: (i, 0, 0)),
        out_shape=jax.ShapeDtypeStruct((B, P, C), jnp.float32),
        compiler_params=pltpu.CompilerParams(
            dimension_semantics=("arbitrary",),
        ),
    )(batch3, grows, brows)


def kernel(batch, labels, gammas, betas):
    B, C, H, W = batch.shape
    labels = labels.astype(jnp.int32)
    grows, brows = _sc_gather_rows(labels, gammas, betas)
    bt = jnp.transpose(batch, (0, 2, 3, 1)).reshape(B, H * W, C)
    out = _film(bt, grows, brows)
    return jnp.transpose(out.reshape(B, H, W, C), (0, 3, 1, 2))
